# baseline (speedup 1.0000x reference)
import functools

import jax
import jax.numpy as jnp
from jax import lax
from jax.experimental import pallas as pl
from jax.experimental.pallas import tpu as pltpu
from jax.experimental.pallas import tpu_sc as plsc

_SB = 128
_UA = 20
_SA = _SB * _UA
_BC = 2560
_GR = 128


def _cumsum_body(seg_ref, x_ref, wn_ref, out_ref, carry_ref, pseg_ref):
    i = pl.program_id(0)

    @pl.when(i == 0)
    def _():
        carry_ref[...] = jnp.zeros_like(carry_ref)
        pseg_ref[...] = jnp.full_like(pseg_ref, -1)

    seg_row = seg_ref[...]
    seg_t = jnp.transpose(seg_row)
    ii = lax.broadcasted_iota(jnp.int32, (_SB, _SB), 0)
    jj = lax.broadcasted_iota(jnp.int32, (_SB, _SB), 1)
    tri = jj <= ii
    wn = wn_ref[...].astype(jnp.bfloat16)
    xb = x_ref[...].astype(jnp.bfloat16)
    carrywn = carry_ref[...]
    prev = pseg_ref[...]
    for s in range(_UA):
        lo = s * _SB
        sr = seg_row[:, lo:lo + _SB]
        sc = seg_t[lo:lo + _SB, :]
        a = (tri & (sr >= sc)).astype(jnp.bfloat16)
        h = lax.dot_general(a, xb[lo:lo + _SB, :],
                            (((1,), (0,)), ((), ())),
                            preferred_element_type=jnp.float32)
        gw = lax.dot_general(h.astype(jnp.bfloat16), wn,
                             (((1,), (1,)), ((), ())),
                             preferred_element_type=jnp.float32)
        mask = (sc == prev).astype(jnp.float32)
        g = gw + mask * carrywn
        out_ref[lo:lo + _SB, :] = g
        carrywn = g[_SB - 1:_SB, :]
        prev = sc[_SB - 1:_SB, :]
    carry_ref[...] = carrywn
    pseg_ref[...] = prev


def _segcumsum_matmul(x, seg32, w_neigh):
    e, d = x.shape
    nb = e // _SA
    return pl.pallas_call(
        _cumsum_body,
        grid=(nb,),
        in_specs=[
            pl.BlockSpec((1, _SA), lambda i: (0, i)),
            pl.BlockSpec((_SA, d), lambda i: (i, 0)),
            pl.BlockSpec((d, d), lambda i: (0, 0)),
        ],
        out_specs=pl.BlockSpec((_SA, d), lambda i: (i, 0)),
        out_shape=jax.ShapeDtypeStruct((e, d), jnp.float32),
        scratch_shapes=[pltpu.VMEM((1, d), jnp.float32),
                        pltpu.VMEM((1, 1), jnp.int32)],
        compiler_params=pltpu.CompilerParams(
            dimension_semantics=("arbitrary",)),
    )(seg32.reshape(1, e), x, w_neigh)


def _sc_gather(g, idx2):
    e, d = g.shape
    n_ops = idx2.shape[0]
    info = plsc.get_sparse_core_info()
    nc, ns = info.num_cores, info.num_subcores
    nw = nc * ns
    nring = 4
    iters = -(-n_ops // nw)
    iters = -(-iters // nring) * nring
    mesh = plsc.VectorSubcoreMesh(core_axis_name="c", subcore_axis_name="s")

    @functools.partial(
        pl.kernel,
        out_type=jax.ShapeDtypeStruct((e, d), jnp.float32),
        mesh=mesh,
        scratch_types=[
            pltpu.VMEM((nring, _GR), jnp.int32),
            [pltpu.VMEM((_GR, d), jnp.float32) for _ in range(nring)],
            [pltpu.SemaphoreType.DMA for _ in range(nring)],
        ],
    )
    def gather_k(g_hbm, idx_hbm, out_hbm, idx_v, rows, sems):
        wid = lax.axis_index("s") * nc + lax.axis_index("c")

        def start(op, b):
            @pl.when(op < n_ops)
            def _():
                pltpu.sync_copy(idx_hbm.at[op], idx_v.at[b])
                pltpu.async_copy(g_hbm.at[idx_v.at[b]], rows[b], sems[b])

        def drain(op, b):
            @pl.when(op < n_ops)
            def _():
                pltpu.make_async_copy(
                    g_hbm.at[pl.ds(0, _GR)], rows[b], sems[b]).wait()
                pltpu.sync_copy(rows[b], out_hbm.at[pl.ds(op * _GR, _GR)])

        for b in range(nring - 1):
            start(b * nw + wid, b)

        def step(k4, c):
            for b in range(nring):
                o = (nring * k4 + b) * nw + wid
                drain(o, b)
                start(o + (nring - 1) * nw, (b + nring - 1) % nring)
            return c

        lax.fori_loop(0, iters // nring, step, 0)

    return gather_k(g, idx2)


def _final_body(dst_ref, hg_ref, deg_ref, ws_ref, bs_ref, bn_ref, out_ref):
    scale = 1.0 / (jnp.transpose(deg_ref[...]) + 1.0)
    t = lax.dot_general(dst_ref[...].astype(jnp.bfloat16),
                        ws_ref[...].astype(jnp.bfloat16),
                        (((1,), (1,)), ((), ())),
                        preferred_element_type=jnp.float32)
    out_ref[...] = t + hg_ref[...] * scale + bs_ref[...] + bn_ref[...]


def _final(dst, hg, deg, w_self, b_self, b_neigh):
    e, d = dst.shape
    nb = e // _BC
    return pl.pallas_call(
        _final_body,
        grid=(nb,),
        in_specs=[
            pl.BlockSpec((_BC, d), lambda i: (i, 0)),
            pl.BlockSpec((_BC, d), lambda i: (i, 0)),
            pl.BlockSpec((1, _BC), lambda i: (0, i)),
            pl.BlockSpec((d, d), lambda i: (0, 0)),
            pl.BlockSpec((1, d), lambda i: (0, 0)),
            pl.BlockSpec((1, d), lambda i: (0, 0)),
        ],
        out_specs=pl.BlockSpec((_BC, d), lambda i: (i, 0)),
        out_shape=jax.ShapeDtypeStruct((e, d), jnp.float32),
        compiler_params=pltpu.CompilerParams(
            dimension_semantics=("parallel",)),
    )(dst, hg, deg.reshape(1, e), w_self, b_self.reshape(1, d),
      b_neigh.reshape(1, d))


def kernel(edge_src_feat, edge_dst_feat, dst_deg, W_self, b_self, W_neigh,
           b_neigh, segment_ids, dst_max_eid, current_layer):
    e, d = edge_src_feat.shape
    seg32 = segment_ids.astype(jnp.int32)
    g = _segcumsum_matmul(edge_src_feat, seg32, W_neigh)
    idx2 = dst_max_eid.astype(jnp.int32).reshape(e // _GR, _GR)
    hg = _sc_gather(g, idx2)
    return _final(edge_dst_feat, hg, dst_deg, W_self, b_self, b_neigh)


# speedup vs baseline: 1.1101x; 1.1101x over previous
import functools

import jax
import jax.numpy as jnp
from jax import lax
from jax.experimental import pallas as pl
from jax.experimental.pallas import tpu as pltpu
from jax.experimental.pallas import tpu_sc as plsc

_SB = 128
_UA = 20
_SA = _SB * _UA
_BC = 2560
_GR = 128


def _cumsum_body(seg_ref, x_ref, wn_ref, out_ref, carry_ref, pseg_ref):
    i = pl.program_id(0)

    @pl.when(i == 0)
    def _():
        carry_ref[...] = jnp.zeros_like(carry_ref)
        pseg_ref[...] = jnp.full_like(pseg_ref, -1)

    seg_row = seg_ref[...]
    seg_t = jnp.transpose(seg_row)
    ii = lax.broadcasted_iota(jnp.int32, (_SB, _SB), 0)
    jj = lax.broadcasted_iota(jnp.int32, (_SB, _SB), 1)
    tri = jj <= ii
    wn = wn_ref[...].astype(jnp.bfloat16)
    xb = x_ref[...].astype(jnp.bfloat16)
    y = lax.dot_general(xb, wn, (((1,), (1,)), ((), ())),
                        preferred_element_type=jnp.float32)
    yb = y.astype(jnp.bfloat16)
    carrywn = carry_ref[...]
    prev = pseg_ref[...]
    for s in range(_UA):
        lo = s * _SB
        sr = seg_row[:, lo:lo + _SB]
        sc = seg_t[lo:lo + _SB, :]
        a = (tri & (sr >= sc)).astype(jnp.bfloat16)
        gw = lax.dot_general(a, yb[lo:lo + _SB, :],
                             (((1,), (0,)), ((), ())),
                             preferred_element_type=jnp.float32)
        mask = (sc == prev).astype(jnp.float32)
        g = gw + mask * carrywn
        out_ref[lo:lo + _SB, :] = g
        carrywn = g[_SB - 1:_SB, :]
        prev = sc[_SB - 1:_SB, :]
    carry_ref[...] = carrywn
    pseg_ref[...] = prev


def _segcumsum_matmul(x, seg32, w_neigh):
    e, d = x.shape
    nb = e // _SA
    return pl.pallas_call(
        _cumsum_body,
        grid=(nb,),
        in_specs=[
            pl.BlockSpec((1, _SA), lambda i: (0, i)),
            pl.BlockSpec((_SA, d), lambda i: (i, 0)),
            pl.BlockSpec((d, d), lambda i: (0, 0)),
        ],
        out_specs=pl.BlockSpec((_SA, d), lambda i: (i, 0)),
        out_shape=jax.ShapeDtypeStruct((e, d), jnp.float32),
        scratch_shapes=[pltpu.VMEM((1, d), jnp.float32),
                        pltpu.VMEM((1, 1), jnp.int32)],
        compiler_params=pltpu.CompilerParams(
            dimension_semantics=("arbitrary",)),
    )(seg32.reshape(1, e), x, w_neigh)


def _sc_gather(g, idx2):
    e, d = g.shape
    n_ops = idx2.shape[0]
    info = plsc.get_sparse_core_info()
    nc, ns = info.num_cores, info.num_subcores
    nw = nc * ns
    nring = 4
    iters = -(-n_ops // nw)
    iters = -(-iters // nring) * nring
    mesh = plsc.VectorSubcoreMesh(core_axis_name="c", subcore_axis_name="s")

    @functools.partial(
        pl.kernel,
        out_type=jax.ShapeDtypeStruct((e, d), jnp.float32),
        mesh=mesh,
        scratch_types=[
            pltpu.VMEM((nring, _GR), jnp.int32),
            [pltpu.VMEM((_GR, d), jnp.float32) for _ in range(nring)],
            [pltpu.SemaphoreType.DMA for _ in range(nring)],
        ],
    )
    def gather_k(g_hbm, idx_hbm, out_hbm, idx_v, rows, sems):
        wid = lax.axis_index("s") * nc + lax.axis_index("c")

        def start(op, b):
            @pl.when(op < n_ops)
            def _():
                pltpu.sync_copy(idx_hbm.at[op], idx_v.at[b])
                pltpu.async_copy(g_hbm.at[idx_v.at[b]], rows[b], sems[b])

        def drain(op, b):
            @pl.when(op < n_ops)
            def _():
                pltpu.make_async_copy(
                    g_hbm.at[pl.ds(0, _GR)], rows[b], sems[b]).wait()
                pltpu.sync_copy(rows[b], out_hbm.at[pl.ds(op * _GR, _GR)])

        for b in range(nring - 1):
            start(b * nw + wid, b)

        def step(k4, c):
            for b in range(nring):
                o = (nring * k4 + b) * nw + wid
                drain(o, b)
                start(o + (nring - 1) * nw, (b + nring - 1) % nring)
            return c

        lax.fori_loop(0, iters // nring, step, 0)

    return gather_k(g, idx2)


def _final_body(dst_ref, hg_ref, deg_ref, ws_ref, bs_ref, bn_ref, out_ref):
    scale = 1.0 / (jnp.transpose(deg_ref[...]) + 1.0)
    t = lax.dot_general(dst_ref[...].astype(jnp.bfloat16),
                        ws_ref[...].astype(jnp.bfloat16),
                        (((1,), (1,)), ((), ())),
                        preferred_element_type=jnp.float32)
    out_ref[...] = t + hg_ref[...] * scale + bs_ref[...] + bn_ref[...]


def _final(dst, hg, deg, w_self, b_self, b_neigh):
    e, d = dst.shape
    nb = e // _BC
    return pl.pallas_call(
        _final_body,
        grid=(nb,),
        in_specs=[
            pl.BlockSpec((_BC, d), lambda i: (i, 0)),
            pl.BlockSpec((_BC, d), lambda i: (i, 0)),
            pl.BlockSpec((1, _BC), lambda i: (0, i)),
            pl.BlockSpec((d, d), lambda i: (0, 0)),
            pl.BlockSpec((1, d), lambda i: (0, 0)),
            pl.BlockSpec((1, d), lambda i: (0, 0)),
        ],
        out_specs=pl.BlockSpec((_BC, d), lambda i: (i, 0)),
        out_shape=jax.ShapeDtypeStruct((e, d), jnp.float32),
        compiler_params=pltpu.CompilerParams(
            dimension_semantics=("parallel",)),
    )(dst, hg, deg.reshape(1, e), w_self, b_self.reshape(1, d),
      b_neigh.reshape(1, d))


def kernel(edge_src_feat, edge_dst_feat, dst_deg, W_self, b_self, W_neigh,
           b_neigh, segment_ids, dst_max_eid, current_layer):
    e, d = edge_src_feat.shape
    seg32 = segment_ids.astype(jnp.int32)
    g = _segcumsum_matmul(edge_src_feat, seg32, W_neigh)
    idx2 = dst_max_eid.astype(jnp.int32).reshape(e // _GR, _GR)
    hg = _sc_gather(g, idx2)
    return _final(edge_dst_feat, hg, dst_deg, W_self, b_self, b_neigh)


# speedup vs baseline: 1.1470x; 1.0333x over previous
import functools

import jax
import jax.numpy as jnp
from jax import lax
from jax.experimental import pallas as pl
from jax.experimental.pallas import tpu as pltpu
from jax.experimental.pallas import tpu_sc as plsc

_SB = 128
_UA = 25
_SA = _SB * _UA
_BC = 2560
_GR = 128


def _cumsum_body(seg_ref, x_ref, wn_ref, out_ref, carry_ref, pseg_ref):
    i = pl.program_id(0)

    @pl.when(i == 0)
    def _():
        carry_ref[...] = jnp.zeros_like(carry_ref)
        pseg_ref[...] = jnp.full_like(pseg_ref, -1)

    seg_row = seg_ref[...]
    seg_t = jnp.transpose(seg_row)
    ii = lax.broadcasted_iota(jnp.int32, (_SB, _SB), 0)
    jj = lax.broadcasted_iota(jnp.int32, (_SB, _SB), 1)
    tri = jj <= ii
    wn = wn_ref[...].astype(jnp.bfloat16)
    xb = x_ref[...].astype(jnp.bfloat16)
    y = lax.dot_general(xb, wn, (((1,), (1,)), ((), ())),
                        preferred_element_type=jnp.float32)
    yb = y.astype(jnp.bfloat16)
    carrywn = carry_ref[...]
    prev = pseg_ref[...]
    for s in range(_UA):
        lo = s * _SB
        sr = seg_row[:, lo:lo + _SB]
        sc = seg_t[lo:lo + _SB, :]
        a = (tri & (sr >= sc)).astype(jnp.bfloat16)
        gw = lax.dot_general(a, yb[lo:lo + _SB, :],
                             (((1,), (0,)), ((), ())),
                             preferred_element_type=jnp.float32)
        mask = (sc == prev).astype(jnp.float32)
        g = gw + mask * carrywn
        out_ref[lo:lo + _SB, :] = g
        carrywn = g[_SB - 1:_SB, :]
        prev = sc[_SB - 1:_SB, :]
    carry_ref[...] = carrywn
    pseg_ref[...] = prev


def _segcumsum_matmul(x, seg32, w_neigh):
    e, d = x.shape
    nb = e // _SA
    return pl.pallas_call(
        _cumsum_body,
        grid=(nb,),
        in_specs=[
            pl.BlockSpec((1, _SA), lambda i: (0, i)),
            pl.BlockSpec((_SA, d), lambda i: (i, 0)),
            pl.BlockSpec((d, d), lambda i: (0, 0)),
        ],
        out_specs=pl.BlockSpec((_SA, d), lambda i: (i, 0)),
        out_shape=jax.ShapeDtypeStruct((e, d), jnp.float32),
        scratch_shapes=[pltpu.VMEM((1, d), jnp.float32),
                        pltpu.VMEM((1, 1), jnp.int32)],
        compiler_params=pltpu.CompilerParams(
            dimension_semantics=("arbitrary",)),
    )(seg32.reshape(1, e), x, w_neigh)


def _sc_gather(g, idx2):
    e, d = g.shape
    n_ops = idx2.shape[0]
    info = plsc.get_sparse_core_info()
    nc, ns = info.num_cores, info.num_subcores
    nw = nc * ns
    nring = 4
    iters = -(-n_ops // nw)
    iters = -(-iters // nring) * nring
    mesh = plsc.VectorSubcoreMesh(core_axis_name="c", subcore_axis_name="s")

    @functools.partial(
        pl.kernel,
        out_type=jax.ShapeDtypeStruct((e, d), jnp.float32),
        mesh=mesh,
        scratch_types=[
            pltpu.VMEM((nring, _GR), jnp.int32),
            [pltpu.VMEM((_GR, d), jnp.float32) for _ in range(nring)],
            [pltpu.SemaphoreType.DMA for _ in range(nring)],
        ],
    )
    def gather_k(g_hbm, idx_hbm, out_hbm, idx_v, rows, sems):
        wid = lax.axis_index("s") * nc + lax.axis_index("c")

        def start(op, b):
            @pl.when(op < n_ops)
            def _():
                pltpu.sync_copy(idx_hbm.at[op], idx_v.at[b])
                pltpu.async_copy(g_hbm.at[idx_v.at[b]], rows[b], sems[b])

        def drain(op, b):
            @pl.when(op < n_ops)
            def _():
                pltpu.make_async_copy(
                    g_hbm.at[pl.ds(0, _GR)], rows[b], sems[b]).wait()
                pltpu.sync_copy(rows[b], out_hbm.at[pl.ds(op * _GR, _GR)])

        for b in range(nring - 1):
            start(b * nw + wid, b)

        def step(k4, c):
            for b in range(nring):
                o = (nring * k4 + b) * nw + wid
                drain(o, b)
                start(o + (nring - 1) * nw, (b + nring - 1) % nring)
            return c

        lax.fori_loop(0, iters // nring, step, 0)

    return gather_k(g, idx2)


def _final_body(dst_ref, hg_ref, deg_ref, ws_ref, bs_ref, bn_ref, out_ref):
    scale = 1.0 / (jnp.transpose(deg_ref[...]) + 1.0)
    t = lax.dot_general(dst_ref[...].astype(jnp.bfloat16),
                        ws_ref[...].astype(jnp.bfloat16),
                        (((1,), (1,)), ((), ())),
                        preferred_element_type=jnp.float32)
    out_ref[...] = t + hg_ref[...] * scale + bs_ref[...] + bn_ref[...]


def _final(dst, hg, deg, w_self, b_self, b_neigh):
    e, d = dst.shape
    nb = e // _BC
    return pl.pallas_call(
        _final_body,
        grid=(nb,),
        in_specs=[
            pl.BlockSpec((_BC, d), lambda i: (i, 0)),
            pl.BlockSpec((_BC, d), lambda i: (i, 0)),
            pl.BlockSpec((1, _BC), lambda i: (0, i)),
            pl.BlockSpec((d, d), lambda i: (0, 0)),
            pl.BlockSpec((1, d), lambda i: (0, 0)),
            pl.BlockSpec((1, d), lambda i: (0, 0)),
        ],
        out_specs=pl.BlockSpec((_BC, d), lambda i: (i, 0)),
        out_shape=jax.ShapeDtypeStruct((e, d), jnp.float32),
        compiler_params=pltpu.CompilerParams(
            dimension_semantics=("parallel",)),
    )(dst, hg, deg.reshape(1, e), w_self, b_self.reshape(1, d),
      b_neigh.reshape(1, d))


def kernel(edge_src_feat, edge_dst_feat, dst_deg, W_self, b_self, W_neigh,
           b_neigh, segment_ids, dst_max_eid, current_layer):
    e, d = edge_src_feat.shape
    seg32 = segment_ids.astype(jnp.int32)
    g = _segcumsum_matmul(edge_src_feat, seg32, W_neigh)
    idx2 = dst_max_eid.astype(jnp.int32).reshape(e // _GR, _GR)
    hg = _sc_gather(g, idx2)
    return _final(edge_dst_feat, hg, dst_deg, W_self, b_self, b_neigh)
